# baseline (speedup 1.0000x reference)
import functools

import jax
import jax.numpy as jnp
from jax import lax
from jax.experimental import pallas as pl
from jax.experimental.pallas import tpu as pltpu
from jax.experimental.pallas import tpu_sc as plsc

VOCAB = 1000000
D = 64
DPAD = 128
B = 4096
SEQ = 200
HALF = SEQ // 2
IPAD = 128
GSL = 104

_info = plsc.get_sparse_core_info()
NC, NS, L = _info.num_cores, _info.num_subcores, _info.num_lanes
NW = NC * NS
SPW = B // NW
NCH = D // L


TBLK = 16384
VPAD = TBLK * ((VOCAB + TBLK - 1) // TBLK)


def _transpose_body(i_ref, o_ref):
    x = i_ref[...]
    eye = (lax.broadcasted_iota(jnp.int32, (DPAD, DPAD), 0)
           == lax.broadcasted_iota(jnp.int32, (DPAD, DPAD), 1)
           ).astype(jnp.float32)
    z = jnp.concatenate([x[:, 0:TBLK // 2], x[:, TBLK // 2:TBLK]], axis=0)
    o_ref[...] = lax.dot_general(z, eye, (((0,), (0,)), ((), ())),
                                 preferred_element_type=jnp.float32)


def _body(idx_hbm, table_hbm, out_hbm, idx_v, rows_v, out_v, sems):
    wid = lax.axis_index("s") * NC + lax.axis_index("c")
    base = wid * SPW

    pltpu.sync_copy(idx_hbm.at[pl.ds(base, SPW)], idx_v)

    def fire(seq, b):
        pltpu.async_copy(table_hbm.at[idx_v.at[seq, 0, pl.ds(0, GSL)]],
                         rows_v.at[b, 0], sems.at[b])
        pltpu.async_copy(table_hbm.at[idx_v.at[seq, 1, pl.ds(0, GSL)]],
                         rows_v.at[b, 1], sems.at[b])

    def drain(seq, b):
        pltpu.make_async_copy(table_hbm.at[idx_v.at[seq, 0, pl.ds(0, GSL)]],
                              rows_v.at[b, 0], sems.at[b]).wait()
        pltpu.make_async_copy(table_hbm.at[idx_v.at[seq, 1, pl.ds(0, GSL)]],
                              rows_v.at[b, 1], sems.at[b]).wait()

    fire(0, 0)

    @pl.loop(0, SPW, step=2)
    def _outer(s0):
        for b in range(2):
            seq = s0 + b
            nxt = seq + 1

            @pl.when(nxt < SPW)
            def _prefetch():
                fire(nxt, 1 - b)

            drain(seq, b)

            def red(r, accs):
                new = []
                for c in range(NCH):
                    a = accs[c]
                    a = a + rows_v[b, 0, r, pl.ds(L * c, L)]
                    a = a + rows_v[b, 1, r, pl.ds(L * c, L)]
                    new.append(a)
                return tuple(new)

            zeros = tuple(jnp.zeros((L,), jnp.float32) for _ in range(NCH))
            accs = lax.fori_loop(0, HALF, red, zeros)
            for c in range(NCH):
                pooled = accs[c] * (1.0 / SEQ)
                out_v[seq, pl.ds(L * c, L)] = 1.0 / (1.0 + jnp.exp(-pooled))

    pltpu.sync_copy(out_v, out_hbm.at[pl.ds(base, SPW)])


@functools.partial(jax.jit, static_argnums=())
def kernel(indices, embedding_weight):
    v = indices.astype(jnp.int32)
    m = (v // TBLK) * TBLK + (v % (TBLK // 2)) * 2 + (v % TBLK) // (TBLK // 2)
    lo = jnp.pad(m[:, :HALF], ((0, 0), (0, IPAD - HALF)))
    hi = jnp.pad(m[:, HALF:], ((0, 0), (0, IPAD - HALF)))
    idx3 = jnp.reshape(jnp.concatenate([lo, hi], axis=1), (B, 2, IPAD))
    tbl_t = jnp.swapaxes(embedding_weight, 0, 1)
    tbl_pairs = pl.pallas_call(
        _transpose_body,
        grid=(VPAD // TBLK,),
        in_specs=[pl.BlockSpec((D, TBLK), lambda g: (0, g))],
        out_specs=pl.BlockSpec((TBLK // 2, DPAD), lambda g: (g, 0)),
        out_shape=jax.ShapeDtypeStruct((VPAD // 2, DPAD), jnp.float32),
    )(tbl_t)
    tbl = jnp.reshape(tbl_pairs, (VPAD, D))
    run = pl.kernel(
        _body,
        mesh=plsc.VectorSubcoreMesh(core_axis_name="c", subcore_axis_name="s"),
        compiler_params=pltpu.CompilerParams(use_tc_tiling_on_sc=False),
        out_type=jax.ShapeDtypeStruct((B, D), jnp.float32),
        scratch_types=[
            pltpu.VMEM((SPW, 2, IPAD), jnp.int32),
            pltpu.VMEM((2, 2, GSL, D), jnp.float32),
            pltpu.VMEM((SPW, D), jnp.float32),
            pltpu.SemaphoreType.DMA((2,)),
        ],
    )
    return run(idx3, tbl)


# speedup vs baseline: 2.7966x; 2.7966x over previous
import functools

import jax
import jax.numpy as jnp
from jax import lax
from jax.experimental import pallas as pl
from jax.experimental.pallas import tpu as pltpu
from jax.experimental.pallas import tpu_sc as plsc

VOCAB = 1000000
D = 64
DPAD = 128
B = 4096
SEQ = 200
HALF = SEQ // 2

_info = plsc.get_sparse_core_info()
NC, NS, L = _info.num_cores, _info.num_subcores, _info.num_lanes
NW = NC * NS
SPW = B // NW
NCH = D // L


TBLK = 16384
VPAD = TBLK * ((VOCAB + TBLK - 1) // TBLK)


def _transpose_body(i_ref, o_ref):
    x = i_ref[...]
    eye = (lax.broadcasted_iota(jnp.int32, (DPAD, DPAD), 0)
           == lax.broadcasted_iota(jnp.int32, (DPAD, DPAD), 1)
           ).astype(jnp.float32)
    z = jnp.concatenate([x[:, 0:TBLK // 2], x[:, TBLK // 2:TBLK]], axis=0)
    o_ref[...] = lax.dot_general(z, eye, (((0,), (0,)), ((), ())),
                                 preferred_element_type=jnp.float32)


def _body(idx_hbm, table_hbm, out_hbm, idx_v, rows_v, out_v, sems):
    wid = lax.axis_index("s") * NC + lax.axis_index("c")
    base = wid * SPW

    pltpu.sync_copy(idx_hbm.at[pl.ds(base, SPW)], idx_v)

    def fire(seq, b):
        pltpu.async_copy(table_hbm.at[idx_v.at[seq, 0]], rows_v.at[b, 0],
                         sems.at[b])
        pltpu.async_copy(table_hbm.at[idx_v.at[seq, 1]], rows_v.at[b, 1],
                         sems.at[b])

    def drain(seq, b):
        pltpu.make_async_copy(table_hbm.at[idx_v.at[seq, 0]],
                              rows_v.at[b, 0], sems.at[b]).wait()
        pltpu.make_async_copy(table_hbm.at[idx_v.at[seq, 1]],
                              rows_v.at[b, 1], sems.at[b]).wait()

    fire(0, 0)

    @pl.loop(0, SPW, step=2)
    def _outer(s0):
        for b in range(2):
            seq = s0 + b
            nxt = seq + 1

            @pl.when(nxt < SPW)
            def _prefetch():
                fire(nxt, 1 - b)

            drain(seq, b)

            def red(r, accs):
                new = []
                for c in range(NCH):
                    a = accs[c]
                    a = a + rows_v[b, 0, r, pl.ds(L * c, L)]
                    a = a + rows_v[b, 1, r, pl.ds(L * c, L)]
                    new.append(a)
                return tuple(new)

            zeros = tuple(jnp.zeros((L,), jnp.float32) for _ in range(NCH))
            accs = lax.fori_loop(0, HALF, red, zeros)
            for c in range(NCH):
                pooled = accs[c] * (1.0 / SEQ)
                out_v[seq, pl.ds(L * c, L)] = 1.0 / (1.0 + jnp.exp(-pooled))

    pltpu.sync_copy(out_v, out_hbm.at[pl.ds(base, SPW)])


@functools.partial(jax.jit, static_argnums=())
def kernel(indices, embedding_weight):
    v = indices.astype(jnp.int32)
    m = (v // TBLK) * TBLK + (v % (TBLK // 2)) * 2 + (v % TBLK) // (TBLK // 2)
    idx3 = jnp.reshape(m, (B, 2, HALF))
    tbl_t = jnp.swapaxes(embedding_weight, 0, 1)
    tbl_pairs = pl.pallas_call(
        _transpose_body,
        grid=(VPAD // TBLK,),
        in_specs=[pl.BlockSpec((D, TBLK), lambda g: (0, g))],
        out_specs=pl.BlockSpec((TBLK // 2, DPAD), lambda g: (g, 0)),
        out_shape=jax.ShapeDtypeStruct((VPAD // 2, DPAD), jnp.float32),
    )(tbl_t)
    tbl = jnp.reshape(tbl_pairs, (VPAD, D))
    run = pl.kernel(
        _body,
        mesh=plsc.VectorSubcoreMesh(core_axis_name="c", subcore_axis_name="s"),
        compiler_params=pltpu.CompilerParams(use_tc_tiling_on_sc=False),
        out_type=jax.ShapeDtypeStruct((B, D), jnp.float32),
        scratch_types=[
            pltpu.VMEM((SPW, 2, HALF), jnp.int32),
            pltpu.VMEM((2, 2, HALF, D), jnp.float32),
            pltpu.VMEM((SPW, D), jnp.float32),
            pltpu.SemaphoreType.DMA((2,)),
        ],
    )
    return run(idx3, tbl)
